# Initial kernel scaffold; baseline (speedup 1.0000x reference)
import jax
import jax.numpy as jnp
from jax.experimental import pallas as pl


def kernel(x, expert_indices, W1, b1, W2, b2):
    raise NotImplementedError("write your pallas kernel here")


# speedup vs baseline: 2.6480x; 2.6480x over previous
import functools

import jax
import jax.numpy as jnp
from jax.experimental import pallas as pl
from jax.experimental.pallas import tpu as pltpu

_NUM_EXPERTS = 8
_BT = 512


def _ffn_body(x_ref, idx_ref, w1_ref, b1_ref, w2_ref, b2_ref, out_ref):
    e = pl.program_id(1)

    xb = x_ref[...].astype(jnp.bfloat16)
    h = jnp.dot(xb, w1_ref[0], preferred_element_type=jnp.float32)
    h = h + b1_ref[0]
    h = 0.5 * h * (1.0 + jax.lax.erf(h * 0.7071067811865476))
    y = jnp.dot(h.astype(jnp.bfloat16), w2_ref[0],
                preferred_element_type=jnp.float32)
    y = y + b2_ref[0]

    count = jnp.sum((idx_ref[...] == e).astype(jnp.float32), axis=-1)
    contrib = count[:, None] * y

    @pl.when(e == 0)
    def _init():
        out_ref[...] = contrib

    @pl.when(e > 0)
    def _acc():
        out_ref[...] += contrib


def kernel(x, expert_indices, W1, b1, W2, b2):
    batch, seq_len, d_model = x.shape
    n_tok = batch * seq_len
    d_ff = W1.shape[-1]
    top_k = expert_indices.shape[-1]

    x_flat = x.reshape(n_tok, d_model)
    idx_flat = expert_indices.reshape(n_tok, top_k).astype(jnp.int32)
    w1b = W1.astype(jnp.bfloat16)
    w2b = W2.astype(jnp.bfloat16)
    b1r = b1.reshape(_NUM_EXPERTS, 1, d_ff)
    b2r = b2.reshape(_NUM_EXPERTS, 1, d_model)

    grid = (n_tok // _BT, _NUM_EXPERTS)
    out = pl.pallas_call(
        _ffn_body,
        grid=grid,
        in_specs=[
            pl.BlockSpec((_BT, d_model), lambda t, e: (t, 0)),
            pl.BlockSpec((_BT, top_k), lambda t, e: (t, 0)),
            pl.BlockSpec((1, d_model, d_ff), lambda t, e: (e, 0, 0)),
            pl.BlockSpec((1, 1, d_ff), lambda t, e: (e, 0, 0)),
            pl.BlockSpec((1, d_ff, d_model), lambda t, e: (e, 0, 0)),
            pl.BlockSpec((1, 1, d_model), lambda t, e: (e, 0, 0)),
        ],
        out_specs=pl.BlockSpec((_BT, d_model), lambda t, e: (t, 0)),
        out_shape=jax.ShapeDtypeStruct((n_tok, d_model), jnp.float32),
        compiler_params=pltpu.CompilerParams(
            dimension_semantics=("parallel", "arbitrary"),
        ),
    )(x_flat, idx_flat, w1b, b1r, w2b, b2r)
    return out.reshape(batch, seq_len, d_model)


# speedup vs baseline: 3.9164x; 1.4790x over previous
import functools

import jax
import jax.numpy as jnp
from jax import lax
from jax.experimental import pallas as pl
from jax.experimental.pallas import tpu as pltpu
from jax.experimental.pallas import tpu_sc as plsc

_NUM_EXPERTS = 8
_BM = 256
_NC, _NS = 2, 16
_NW = _NC * _NS


def _routing(idx_flat, n_tok, top_k):
    S = n_tok * top_k
    P = S + _NUM_EXPERTS * _BM
    G = P // _BM
    e = idx_flat.reshape(-1).astype(jnp.int32)
    onehot = (e[:, None] == jnp.arange(_NUM_EXPERTS)[None, :]).astype(jnp.int32)
    cum = jnp.cumsum(onehot, axis=0)
    cnt = cum[-1]
    rank = jnp.sum(onehot * cum, axis=1) - 1
    padded = ((cnt + _BM - 1) // _BM) * _BM
    start = jnp.concatenate([jnp.zeros((1,), jnp.int32),
                             jnp.cumsum(padded)[:-1].astype(jnp.int32)])
    q = start[e] + rank
    src = jnp.zeros((P,), jnp.int32).at[q].set(
        jnp.arange(S, dtype=jnp.int32) // top_k)
    blk = jnp.sum(start[None, :] <= (jnp.arange(G, dtype=jnp.int32)[:, None] * _BM),
                  axis=1).astype(jnp.int32) - 1
    q2 = q.reshape(n_tok, top_k)
    return src, blk, q2[:, 0], q2[:, 1], P, G


def _make_gather(P, D):
    rows_w = P // _NW
    CH = 64
    n_ch = rows_w // CH
    mesh = plsc.VectorSubcoreMesh(core_axis_name="c", subcore_axis_name="s")

    @functools.partial(
        pl.kernel,
        out_type=jax.ShapeDtypeStruct((P, D), jnp.float32),
        mesh=mesh,
        scratch_types=[
            pltpu.VMEM((CH,), jnp.int32),
            pltpu.VMEM((CH, D), jnp.float32),
            pltpu.SemaphoreType.DMA,
        ],
    )
    def gather_k(x_hbm, src_hbm, xs_hbm, idx_v, rows_v, sem):
        wid = lax.axis_index("s") * _NC + lax.axis_index("c")
        base = wid * rows_w

        @pl.loop(0, n_ch)
        def _chunk(c):
            off = base + c * CH
            pltpu.sync_copy(src_hbm.at[pl.ds(off, CH)], idx_v)
            pltpu.async_copy(x_hbm.at[idx_v], rows_v, sem).wait()
            pltpu.sync_copy(rows_v, xs_hbm.at[pl.ds(off, CH)])

    return gather_k


def _mm_body(be_ref, xs_ref, w1_ref, b1_ref, w2_ref, b2_ref, out_ref):
    xb = xs_ref[...].astype(jnp.bfloat16)
    h = jnp.dot(xb, w1_ref[0], preferred_element_type=jnp.float32)
    h = h + b1_ref[0]
    h = 0.5 * h * (1.0 + jax.lax.erf(h * 0.7071067811865476))
    y = jnp.dot(h.astype(jnp.bfloat16), w2_ref[0],
                preferred_element_type=jnp.float32)
    out_ref[...] = y + b2_ref[0]


def _make_combine(P, D, n_tok):
    toks_w = n_tok // _NW
    CHT = 32
    n_ch = toks_w // CHT
    n_vec = CHT * D // 16
    mesh = plsc.VectorSubcoreMesh(core_axis_name="c", subcore_axis_name="s")

    @functools.partial(
        pl.kernel,
        out_type=jax.ShapeDtypeStruct((n_tok, D), jnp.float32),
        mesh=mesh,
        scratch_types=[
            pltpu.VMEM((CHT,), jnp.int32),
            pltpu.VMEM((CHT,), jnp.int32),
            pltpu.VMEM((CHT, D), jnp.float32),
            pltpu.VMEM((CHT, D), jnp.float32),
            pltpu.SemaphoreType.DMA,
            pltpu.SemaphoreType.DMA,
        ],
    )
    def combine_k(ys_hbm, qe_hbm, qo_hbm, out_hbm,
                  idx_a, idx_b, rows_a, rows_b, sem_a, sem_b):
        wid = lax.axis_index("s") * _NC + lax.axis_index("c")
        base = wid * toks_w

        @pl.loop(0, n_ch)
        def _chunk(c):
            off = base + c * CHT
            pltpu.sync_copy(qe_hbm.at[pl.ds(off, CHT)], idx_a)
            pltpu.sync_copy(qo_hbm.at[pl.ds(off, CHT)], idx_b)
            cp_a = pltpu.async_copy(ys_hbm.at[idx_a], rows_a, sem_a)
            cp_b = pltpu.async_copy(ys_hbm.at[idx_b], rows_b, sem_b)
            cp_a.wait()
            cp_b.wait()

            @pl.loop(0, CHT)
            def _row(i):
                @pl.loop(0, D // 16, unroll=8)
                def _vec(v):
                    s = pl.ds(v * 16, 16)
                    rows_a[i, s] = rows_a[i, s] + rows_b[i, s]

            pltpu.sync_copy(rows_a, out_hbm.at[pl.ds(off, CHT)])

    return combine_k


def kernel(x, expert_indices, W1, b1, W2, b2):
    batch, seq_len, d_model = x.shape
    n_tok = batch * seq_len
    d_ff = W1.shape[-1]
    top_k = expert_indices.shape[-1]

    x_flat = x.reshape(n_tok, d_model)
    idx_flat = expert_indices.reshape(n_tok, top_k).astype(jnp.int32)
    w1b = W1.astype(jnp.bfloat16)
    w2b = W2.astype(jnp.bfloat16)
    b1r = b1.reshape(_NUM_EXPERTS, 1, d_ff)
    b2r = b2.reshape(_NUM_EXPERTS, 1, d_model)

    src, blk, qe, qo, P, G = _routing(idx_flat, n_tok, top_k)

    xs = _make_gather(P, d_model)(x_flat, src)

    grid_spec = pltpu.PrefetchScalarGridSpec(
        num_scalar_prefetch=1,
        grid=(G,),
        in_specs=[
            pl.BlockSpec((_BM, d_model), lambda g, be: (g, 0)),
            pl.BlockSpec((1, d_model, d_ff), lambda g, be: (be[g], 0, 0)),
            pl.BlockSpec((1, 1, d_ff), lambda g, be: (be[g], 0, 0)),
            pl.BlockSpec((1, d_ff, d_model), lambda g, be: (be[g], 0, 0)),
            pl.BlockSpec((1, 1, d_model), lambda g, be: (be[g], 0, 0)),
        ],
        out_specs=pl.BlockSpec((_BM, d_model), lambda g, be: (g, 0)),
    )
    ys = pl.pallas_call(
        _mm_body,
        grid_spec=grid_spec,
        out_shape=jax.ShapeDtypeStruct((P, d_model), jnp.float32),
        compiler_params=pltpu.CompilerParams(
            dimension_semantics=("arbitrary",),
        ),
    )(blk, xs, w1b, b1r, w2b, b2r)

    out = _make_combine(P, d_model, n_tok)(ys, qe, qo)
    return out.reshape(batch, seq_len, d_model)
